# baseline (speedup 1.0000x reference)
import functools

import jax
import jax.numpy as jnp
from jax.experimental import pallas as pl
from jax.experimental.pallas import tpu as pltpu

_B = 8
_H = 1024
_M = 65536
_TILE = 2048
_EPS = 1e-12


def _body(hs_ref, mem_ref, ln_g_ref, ln_b_ref, wq_ref, bq_ref, wo_ref, bo_ref,
          out_ref, q_scr, m_scr, l_scr, acc_scr):
    i = pl.program_id(0)

    @pl.when(i == 0)
    def _prologue():
        x = hs_ref[...]
        mu = jnp.mean(x, axis=-1, keepdims=True)
        var = jnp.mean((x - mu) ** 2, axis=-1, keepdims=True)
        nh = (x - mu) / jnp.sqrt(var + _EPS) * ln_g_ref[...] + ln_b_ref[...]
        q = jax.lax.dot_general(nh, wq_ref[...], (((1,), (1,)), ((), ())),
                                preferred_element_type=jnp.float32)
        q_scr[...] = q + bq_ref[...]
        m_scr[...] = jnp.full((_B, 1), -jnp.inf, dtype=jnp.float32)
        l_scr[...] = jnp.zeros((_B, 1), dtype=jnp.float32)
        acc_scr[...] = jnp.zeros((_B, _H), dtype=jnp.float32)

    tile = mem_ref[...]
    s = jax.lax.dot_general(q_scr[...], tile, (((1,), (1,)), ((), ())),
                            preferred_element_type=jnp.float32)
    m_prev = m_scr[...]
    m_new = jnp.maximum(m_prev, jnp.max(s, axis=-1, keepdims=True))
    p = jnp.exp(s - m_new)
    scale = jnp.exp(m_prev - m_new)
    l_scr[...] = l_scr[...] * scale + jnp.sum(p, axis=-1, keepdims=True)
    pv = jax.lax.dot_general(p, tile, (((1,), (0,)), ((), ())),
                             preferred_element_type=jnp.float32)
    acc_scr[...] = acc_scr[...] * scale + pv
    m_scr[...] = m_new

    @pl.when(i == pl.num_programs(0) - 1)
    def _epilogue():
        mo = acc_scr[...] / l_scr[...]
        proj = jax.lax.dot_general(mo, wo_ref[...], (((1,), (1,)), ((), ())),
                                   preferred_element_type=jnp.float32)
        out_ref[...] = hs_ref[...] + proj + bo_ref[...]


def kernel(hidden_states, memory, ln_g, ln_b, Wq, bq, Wo, bo):
    hs = hidden_states.reshape(_B, _H)
    mem = memory.reshape(_M, _H)
    grid = (_M // _TILE,)

    def full(shape):
        return pl.BlockSpec(shape, lambda *_: tuple(0 for _ in shape))

    out = pl.pallas_call(
        _body,
        grid=grid,
        in_specs=[
            full((_B, _H)),
            pl.BlockSpec((_TILE, _H), lambda i: (i, 0)),
            full((_H,)), full((_H,)),
            full((_H, _H)), full((_H,)),
            full((_H, _H)), full((_H,)),
        ],
        out_specs=full((_B, _H)),
        out_shape=jax.ShapeDtypeStruct((_B, _H), jnp.float32),
        scratch_shapes=[
            pltpu.VMEM((_B, _H), jnp.float32),
            pltpu.VMEM((_B, 1), jnp.float32),
            pltpu.VMEM((_B, 1), jnp.float32),
            pltpu.VMEM((_B, _H), jnp.float32),
        ],
    )(hs, mem, ln_g, ln_b, Wq, bq, Wo, bo)
    return out.reshape(_B, 1, _H)


# speedup vs baseline: 1.0880x; 1.0880x over previous
import functools

import jax
import jax.numpy as jnp
from jax.experimental import pallas as pl
from jax.experimental.pallas import tpu as pltpu

_B = 8
_H = 1024
_M = 65536
_TILE = 4096
_EPS = 1e-12


def _body(hs_ref, mem_ref, ln_g_ref, ln_b_ref, wq_ref, bq_ref, wo_ref, bo_ref,
          out_ref, q_scr, m_scr, l_scr, acc_scr):
    i = pl.program_id(0)

    @pl.when(i == 0)
    def _prologue():
        x = hs_ref[...]
        mu = jnp.mean(x, axis=-1, keepdims=True)
        var = jnp.mean((x - mu) ** 2, axis=-1, keepdims=True)
        nh = (x - mu) / jnp.sqrt(var + _EPS) * ln_g_ref[...] + ln_b_ref[...]
        q = jax.lax.dot_general(nh, wq_ref[...], (((1,), (1,)), ((), ())),
                                preferred_element_type=jnp.float32)
        q_scr[...] = q + bq_ref[...]
        m_scr[...] = jnp.full((_B, 1), -jnp.inf, dtype=jnp.float32)
        l_scr[...] = jnp.zeros((_B, 1), dtype=jnp.float32)
        acc_scr[...] = jnp.zeros((_B, _H), dtype=jnp.float32)

    tile = mem_ref[...]
    s = jax.lax.dot_general(q_scr[...], tile, (((1,), (1,)), ((), ())),
                            preferred_element_type=jnp.float32)
    m_prev = m_scr[...]
    m_new = jnp.maximum(m_prev, jnp.max(s, axis=-1, keepdims=True))
    p = jnp.exp(s - m_new)
    scale = jnp.exp(m_prev - m_new)
    l_scr[...] = l_scr[...] * scale + jnp.sum(p, axis=-1, keepdims=True)
    pv = jax.lax.dot_general(p, tile, (((1,), (0,)), ((), ())),
                             preferred_element_type=jnp.float32)
    acc_scr[...] = acc_scr[...] * scale + pv
    m_scr[...] = m_new

    @pl.when(i == pl.num_programs(0) - 1)
    def _epilogue():
        mo = acc_scr[...] / l_scr[...]
        proj = jax.lax.dot_general(mo, wo_ref[...], (((1,), (1,)), ((), ())),
                                   preferred_element_type=jnp.float32)
        out_ref[...] = hs_ref[...] + proj + bo_ref[...]


def kernel(hidden_states, memory, ln_g, ln_b, Wq, bq, Wo, bo):
    hs = hidden_states.reshape(_B, _H)
    mem = memory.reshape(_M, _H)
    grid = (_M // _TILE,)

    def full(shape):
        return pl.BlockSpec(shape, lambda *_: tuple(0 for _ in shape))

    out = pl.pallas_call(
        _body,
        grid=grid,
        in_specs=[
            full((_B, _H)),
            pl.BlockSpec((_TILE, _H), lambda i: (i, 0)),
            full((_H,)), full((_H,)),
            full((_H, _H)), full((_H,)),
            full((_H, _H)), full((_H,)),
        ],
        out_specs=full((_B, _H)),
        out_shape=jax.ShapeDtypeStruct((_B, _H), jnp.float32),
        scratch_shapes=[
            pltpu.VMEM((_B, _H), jnp.float32),
            pltpu.VMEM((_B, 1), jnp.float32),
            pltpu.VMEM((_B, 1), jnp.float32),
            pltpu.VMEM((_B, _H), jnp.float32),
        ],
    )(hs, mem, ln_g, ln_b, Wq, bq, Wo, bo)
    return out.reshape(_B, 1, _H)
